# baseline (speedup 1.0000x reference)
import jax
import jax.numpy as jnp
from jax.experimental import pallas as pl
from jax.experimental.pallas import tpu as pltpu

_NUM_STEPS = 1000
_BETA_START = 0.0001
_BETA_END = 0.02
_TAB = 1024
_BB = 32


def _tables():
    betas = jnp.linspace(_BETA_START, _BETA_END, _NUM_STEPS, dtype=jnp.float32)
    ac = jnp.cumprod(1.0 - betas)
    a = jnp.sqrt(ac)
    s = jnp.sqrt(1.0 - ac)
    pad = (0, _TAB - _NUM_STEPS)
    return jnp.pad(a, pad).reshape(1, _TAB), jnp.pad(s, pad).reshape(1, _TAB)


def _scale_body(t_ref, a_ref, s_ref, x_ref, n_ref, o_ref):
    tv = t_ref[...]
    iota = jax.lax.broadcasted_iota(jnp.int32, (_BB, _TAB), 1)
    m = iota == tv
    a = jnp.sum(jnp.where(m, a_ref[...], 0.0), axis=1).reshape(_BB, 1, 1)
    s = jnp.sum(jnp.where(m, s_ref[...], 0.0), axis=1).reshape(_BB, 1, 1)
    o_ref[...] = a * x_ref[...] + s * n_ref[...]


def kernel(x0, t, noise):
    B, C, W = x0.shape
    a_tab, s_tab = _tables()
    t2 = t.reshape(B, 1)
    out = pl.pallas_call(
        _scale_body,
        grid=(B // _BB,),
        in_specs=[
            pl.BlockSpec((_BB, 1), lambda i: (i, 0)),
            pl.BlockSpec((1, _TAB), lambda i: (0, 0)),
            pl.BlockSpec((1, _TAB), lambda i: (0, 0)),
            pl.BlockSpec((_BB, C, W), lambda i: (i, 0, 0)),
            pl.BlockSpec((_BB, C, W), lambda i: (i, 0, 0)),
        ],
        out_specs=pl.BlockSpec((_BB, C, W), lambda i: (i, 0, 0)),
        out_shape=jax.ShapeDtypeStruct((B, C, W), jnp.float32),
        compiler_params=pltpu.CompilerParams(dimension_semantics=("parallel",)),
    )(t2, a_tab, s_tab, x0, noise)
    return out
